# baseline (speedup 1.0000x reference)
import functools

import jax
import jax.numpy as jnp
from jax import lax
from jax.experimental import pallas as pl
from jax.experimental.pallas import tpu as pltpu
from jax.experimental.pallas import tpu_sc as plsc

_S_TC = 1792
_CHUNK = 32
_L = 16
_DB = 16


def _tc_body(ids_ref, x_ref, out_ref):
    w = (ids_ref[0] != 0).astype(jnp.float32)
    out_ref[0] = jnp.sum(x_ref[0] * w, axis=0, keepdims=True)


def _sc_body(s_total, s_tc, d, x_hbm, w_hbm, out_hbm,
             buf, wtile, acc, sem0, sem1):
    rows_per_tile = (s_total - s_tc) // 2
    nchunks = rows_per_tile // _CHUNK
    wid = lax.axis_index("s") * 2 + lax.axis_index("c")
    b = wid // 2
    half = wid % 2
    row0 = s_tc + half * rows_per_tile

    pltpu.sync_copy(w_hbm.at[b, pl.ds(half * rows_per_tile, rows_per_tile)],
                    wtile)

    sems = (sem0, sem1)
    cps = [None, None]
    cps[0] = pltpu.async_copy(
        x_hbm.at[b, pl.ds(row0, _CHUNK)], buf.at[0], sems[0])
    zero = jnp.zeros((_L,), jnp.float32)
    for c in range(nchunks):
        slot = c % 2
        if c + 1 < nchunks:
            nslot = (c + 1) % 2
            cps[nslot] = pltpu.async_copy(
                x_hbm.at[b, pl.ds(row0 + (c + 1) * _CHUNK, _CHUNK)],
                buf.at[nslot], sems[nslot])
        cps[slot].wait()

        for db in range(d // _L // _DB):
            def row_body(r, accs, c=c, slot=slot, db=db):
                wf = wtile[c * _CHUNK + r]
                return tuple(
                    a + buf[slot, r, pl.ds((db * _DB + k) * _L, _L)] * wf
                    for k, a in enumerate(accs))

            accs = plsc.parallel_loop(
                0, _CHUNK, carry=(zero,) * _DB, unroll=2)(row_body)
            for k in range(_DB):
                sl = pl.ds((db * _DB + k) * _L, _L)
                if c == 0:
                    acc[sl] = accs[k]
                else:
                    plsc.addupdate(acc.at[sl], accs[k])

    pltpu.sync_copy(acc, out_hbm.at[b, half])


def _combine_body(ids_ref, tc_ref, sc_ref, out_ref):
    w = (ids_ref[:, :, 0] != 0).astype(jnp.float32)
    cnt = jnp.sum(w, axis=1, keepdims=True)
    tot = tc_ref[:, 0, :] + sc_ref[:, 0, :] + sc_ref[:, 1, :]
    out_ref[...] = tot / cnt


def kernel(x, input):
    B, S, D = x.shape
    ids = input.astype(jnp.int32)
    ids3 = ids.reshape(B, S, 1)

    tc_sum = pl.pallas_call(
        _tc_body,
        grid=(B,),
        in_specs=[
            pl.BlockSpec((1, _S_TC, 1), lambda b: (b, 0, 0)),
            pl.BlockSpec((1, _S_TC, D), lambda b: (b, 0, 0)),
        ],
        out_specs=pl.BlockSpec((1, 1, D), lambda b: (b, 0, 0)),
        out_shape=jax.ShapeDtypeStruct((B, 1, D), jnp.float32),
    )(ids3, x)

    mesh = plsc.VectorSubcoreMesh(core_axis_name="c", subcore_axis_name="s",
                                  num_cores=2, num_subcores=16)
    sc_fn = pl.kernel(
        functools.partial(_sc_body, S, _S_TC, D),
        out_type=jax.ShapeDtypeStruct((B, 2, D), jnp.float32),
        mesh=mesh,
        scratch_types=[
            pltpu.VMEM((2, _CHUNK, D), jnp.float32),
            pltpu.VMEM(((S - _S_TC) // 2, _L), jnp.float32),
            pltpu.VMEM((D,), jnp.float32),
            pltpu.SemaphoreType.DMA,
            pltpu.SemaphoreType.DMA,
        ],
    )
    s_sc = S - _S_TC
    w_exp = jnp.broadcast_to(
        (ids[:, _S_TC:, None] != 0).astype(jnp.float32), (B, s_sc, _L))
    sc_sum = sc_fn(x, w_exp)

    out = pl.pallas_call(
        _combine_body,
        in_specs=[
            pl.BlockSpec((B, S, 1), lambda: (0, 0, 0)),
            pl.BlockSpec((B, 1, D), lambda: (0, 0, 0)),
            pl.BlockSpec((B, 2, D), lambda: (0, 0, 0)),
        ],
        out_specs=pl.BlockSpec((B, D), lambda: (0, 0)),
        out_shape=jax.ShapeDtypeStruct((B, D), jnp.float32),
    )(ids3, tc_sum, sc_sum)
    return out


# speedup vs baseline: 1.3129x; 1.3129x over previous
import functools

import jax
import jax.numpy as jnp
from jax import lax
from jax.experimental import pallas as pl
from jax.experimental.pallas import tpu as pltpu
from jax.experimental.pallas import tpu_sc as plsc

_S_TC = 1792
_CHUNK = 32
_L = 16
_DB = 16


def _tc_body(ids_ref, x_ref, out_ref):
    w = (ids_ref[0] != 0).astype(jnp.float32)
    out_ref[0] = jax.lax.dot_general(
        w, x_ref[0], (((1,), (0,)), ((), ())),
        preferred_element_type=jnp.float32)


def _sc_body(s_total, s_tc, d, x_hbm, w_hbm, out_hbm,
             buf, wtile, acc, sem0, sem1):
    rows_per_tile = (s_total - s_tc) // 2
    nchunks = rows_per_tile // _CHUNK
    wid = lax.axis_index("s") * 2 + lax.axis_index("c")
    b = wid // 2
    half = wid % 2
    row0 = s_tc + half * rows_per_tile

    pltpu.sync_copy(w_hbm.at[b, pl.ds(half * rows_per_tile, rows_per_tile)],
                    wtile)

    sems = (sem0, sem1)
    cps = [None, None]
    cps[0] = pltpu.async_copy(
        x_hbm.at[b, pl.ds(row0, _CHUNK)], buf.at[0], sems[0])
    zero = jnp.zeros((_L,), jnp.float32)
    for c in range(nchunks):
        slot = c % 2
        if c + 1 < nchunks:
            nslot = (c + 1) % 2
            cps[nslot] = pltpu.async_copy(
                x_hbm.at[b, pl.ds(row0 + (c + 1) * _CHUNK, _CHUNK)],
                buf.at[nslot], sems[nslot])
        cps[slot].wait()

        for db in range(d // _L // _DB):
            def row_body(r, accs, c=c, slot=slot, db=db):
                wf = wtile[c * _CHUNK + r]
                return tuple(
                    a + buf[slot, r, pl.ds((db * _DB + k) * _L, _L)] * wf
                    for k, a in enumerate(accs))

            accs = plsc.parallel_loop(
                0, _CHUNK, carry=(zero,) * _DB, unroll=2)(row_body)
            for k in range(_DB):
                sl = pl.ds((db * _DB + k) * _L, _L)
                if c == 0:
                    acc[sl] = accs[k]
                else:
                    plsc.addupdate(acc.at[sl], accs[k])

    pltpu.sync_copy(acc, out_hbm.at[b, half])


def _combine_body(ids_ref, tc_ref, sc_ref, out_ref):
    w = (ids_ref[:, 0, :] != 0).astype(jnp.float32)
    cnt = jnp.sum(w, axis=1, keepdims=True)
    tot = tc_ref[:, 0, :] + sc_ref[:, 0, :] + sc_ref[:, 1, :]
    out_ref[...] = tot / cnt


def kernel(x, input):
    B, S, D = x.shape
    ids = input.astype(jnp.int32)
    ids3 = ids.reshape(B, 1, S)

    tc_sum = pl.pallas_call(
        _tc_body,
        grid=(B,),
        in_specs=[
            pl.BlockSpec((1, 1, _S_TC), lambda b: (b, 0, 0)),
            pl.BlockSpec((1, _S_TC, D), lambda b: (b, 0, 0)),
        ],
        out_specs=pl.BlockSpec((1, 1, D), lambda b: (b, 0, 0)),
        out_shape=jax.ShapeDtypeStruct((B, 1, D), jnp.float32),
    )(ids3, x)

    mesh = plsc.VectorSubcoreMesh(core_axis_name="c", subcore_axis_name="s",
                                  num_cores=2, num_subcores=16)
    sc_fn = pl.kernel(
        functools.partial(_sc_body, S, _S_TC, D),
        out_type=jax.ShapeDtypeStruct((B, 2, D), jnp.float32),
        mesh=mesh,
        scratch_types=[
            pltpu.VMEM((2, _CHUNK, D), jnp.float32),
            pltpu.VMEM(((S - _S_TC) // 2, _L), jnp.float32),
            pltpu.VMEM((D,), jnp.float32),
            pltpu.SemaphoreType.DMA,
            pltpu.SemaphoreType.DMA,
        ],
    )
    s_sc = S - _S_TC
    w_exp = jnp.broadcast_to(
        (ids[:, _S_TC:, None] != 0).astype(jnp.float32), (B, s_sc, _L))
    sc_sum = sc_fn(x, w_exp)

    out = pl.pallas_call(
        _combine_body,
        in_specs=[
            pl.BlockSpec((B, 1, S), lambda: (0, 0, 0)),
            pl.BlockSpec((B, 1, D), lambda: (0, 0, 0)),
            pl.BlockSpec((B, 2, D), lambda: (0, 0, 0)),
        ],
        out_specs=pl.BlockSpec((B, D), lambda: (0, 0)),
        out_shape=jax.ShapeDtypeStruct((B, D), jnp.float32),
    )(ids3, tc_sum, sc_sum)
    return out


# speedup vs baseline: 1.6133x; 1.2288x over previous
import jax
import jax.numpy as jnp
from jax.experimental import pallas as pl

_NJ = 2


def _body(ids_ref, idchunk_ref, x_ref, out_ref):
    j = pl.program_id(1)
    w = (idchunk_ref[0] != 0).astype(jnp.float32)
    s = jax.lax.dot_general(
        w, x_ref[0], (((1,), (0,)), ((), ())),
        preferred_element_type=jnp.float32)

    @pl.when(j == 0)
    def _init():
        out_ref[0] = s

    @pl.when(j == _NJ - 1)
    def _fin():
        cnt = jnp.sum((ids_ref[0] != 0).astype(jnp.float32))
        out_ref[0] = (out_ref[0] + s) / cnt


def kernel(x, input):
    B, S, D = x.shape
    sc = S // _NJ
    ids3 = input.reshape(B, 1, S).astype(jnp.int32)
    out = pl.pallas_call(
        _body,
        grid=(B, _NJ),
        in_specs=[
            pl.BlockSpec((1, 1, S), lambda b, j: (b, 0, 0)),
            pl.BlockSpec((1, 1, sc), lambda b, j: (b, 0, j)),
            pl.BlockSpec((1, sc, D), lambda b, j: (b, j, 0)),
        ],
        out_specs=pl.BlockSpec((1, 1, D), lambda b, j: (b, 0, 0)),
        out_shape=jax.ShapeDtypeStruct((B, 1, D), jnp.float32),
    )(ids3, ids3, x)
    return out.reshape(B, D)


# speedup vs baseline: 1.8333x; 1.1364x over previous
import jax
import jax.numpy as jnp
from jax.experimental import pallas as pl


def _body(ids_ref, x_ref, out_ref):
    w = (ids_ref[0] != 0).astype(jnp.bfloat16)
    s = jax.lax.dot_general(
        w, x_ref[0].astype(jnp.bfloat16), (((1,), (0,)), ((), ())),
        preferred_element_type=jnp.float32)
    cnt = jnp.sum((ids_ref[0] != 0).astype(jnp.float32))
    out_ref[0] = s / cnt


def kernel(x, input):
    B, S, D = x.shape
    ids3 = input.reshape(B, 1, S).astype(jnp.int32)
    out = pl.pallas_call(
        _body,
        grid=(B,),
        in_specs=[
            pl.BlockSpec((1, 1, S), lambda b: (b, 0, 0)),
            pl.BlockSpec((1, S, D), lambda b: (b, 0, 0)),
        ],
        out_specs=pl.BlockSpec((1, 1, D), lambda b: (b, 0, 0)),
        out_shape=jax.ShapeDtypeStruct((B, 1, D), jnp.float32),
    )(ids3, x)
    return out.reshape(B, D)
